# Initial kernel scaffold; baseline (speedup 1.0000x reference)
import jax
import jax.numpy as jnp
from jax.experimental import pallas as pl


def kernel(x):
    raise NotImplementedError("write your pallas kernel here")


# speedup vs baseline: 1.4543x; 1.4543x over previous
import functools

import jax
import jax.numpy as jnp
from jax import lax
from jax.experimental import pallas as pl
from jax.experimental.pallas import tpu as pltpu
from jax.experimental.pallas import tpu_sc as plsc

NUM_KEEP = 2048

_info = plsc.get_sparse_core_info()
_NC, _NS = _info.num_cores, _info.num_subcores
_NW = _NC * _NS


@functools.lru_cache(maxsize=None)
def _make_gather(R, D, rows_per_w, chunk):
    nchunks = rows_per_w // chunk
    mesh = plsc.VectorSubcoreMesh(core_axis_name="c", subcore_axis_name="s")

    @functools.partial(
        pl.kernel,
        mesh=mesh,
        out_type=jax.ShapeDtypeStruct((R, D), jnp.float32),
        scratch_types=[
            pltpu.VMEM((nchunks, chunk), jnp.int32),
            pltpu.VMEM((chunk, D), jnp.float32),
            pltpu.SemaphoreType.DMA,
        ],
    )
    def gather_k(x_hbm, gidx_hbm, out_hbm, idx_v, rows_v, sem):
        wid = lax.axis_index("s") * _NC + lax.axis_index("c")
        base = wid * rows_per_w
        pltpu.sync_copy(gidx_hbm.at[wid], idx_v)
        for j in range(nchunks):
            pltpu.async_copy(x_hbm.at[idx_v.at[j]], rows_v, sem).wait()
            pltpu.sync_copy(rows_v, out_hbm.at[pl.ds(base + j * chunk, chunk)])

    return gather_k


def kernel(x):
    B, N, D = x.shape
    noise = jax.random.uniform(jax.random.key(1), (B, N), dtype=jnp.float32)
    ids = jnp.argsort(noise, axis=1)[:, :NUM_KEEP]
    gidx = (ids + (jnp.arange(B, dtype=ids.dtype) * N)[:, None]).astype(jnp.int32)

    R = B * NUM_KEEP
    rows_per_w = R // _NW
    chunk = 64
    gidx = gidx.reshape(_NW, rows_per_w // chunk, chunk)
    out = _make_gather(R, D, rows_per_w, chunk)(x.reshape(B * N, D), gidx)
    return out.reshape(B, NUM_KEEP, D)
